# baseline (speedup 1.0000x reference)
import functools

import jax
import jax.numpy as jnp
from jax import lax
from jax.experimental import pallas as pl
from jax.experimental.pallas import tpu as pltpu
from jax.experimental.pallas import tpu_sc as plsc

_NC = 2
_NS = 16
_NW = _NC * _NS
_LANES = 16
_IDX_W = 128


def _make_pool(V, D, B, LIN):
    items_per_w = B // _NW
    n_groups = items_per_w // _IDX_W
    n_steps = n_groups * LIN

    mesh = plsc.VectorSubcoreMesh(core_axis_name="c", subcore_axis_name="s")

    @functools.partial(
        pl.kernel,
        out_type=jax.ShapeDtypeStruct((B, D), jnp.float32),
        mesh=mesh,
        compiler_params=pltpu.CompilerParams(use_tc_tiling_on_sc=False),
        scratch_types=[
            pltpu.VMEM((LIN, items_per_w), jnp.int32),
            pltpu.VMEM((_IDX_W, D), jnp.float32),
            pltpu.VMEM((_IDX_W, D), jnp.float32),
            pltpu.VMEM((_IDX_W, D), jnp.float32),
            pltpu.VMEM((_IDX_W, D), jnp.float32),
            pltpu.SemaphoreType.DMA,
            pltpu.SemaphoreType.DMA,
            pltpu.SemaphoreType.DMA,
            pltpu.SemaphoreType.DMA,
        ],
    )
    def pool(table_h, idx_h, out_h, idx_v, slab0, slab1, outv0, outv1,
             sem_s0, sem_s1, sem_o0, sem_o1):
        wid = lax.axis_index("s") * _NC + lax.axis_index("c")
        pltpu.sync_copy(idx_h.at[:, pl.ds(wid * items_per_w, items_per_w)],
                        idx_v)

        slabs = (slab0, slab1)
        slab_sems = (sem_s0, sem_s1)
        outvs = (outv0, outv1)
        out_sems = (sem_o0, sem_o1)

        def start_gather(s):
            g, j = s // LIN, s % LIN
            p = s % 2
            return pltpu.async_copy(
                table_h.at[idx_v.at[j, pl.ds(g * _IDX_W, _IDX_W)]],
                slabs[p], slab_sems[p])

        def accumulate(s):
            g, j = s // LIN, s % LIN
            slab = slabs[s % 2]
            outv = outvs[g % 2]

            @pl.loop(0, _IDX_W, unroll=2)
            def _(i):
                for t in range(D // _LANES):
                    sl = pl.ds(t * _LANES, _LANES)
                    if j == 0:
                        outv[i, sl] = slab[i, sl]
                    else:
                        outv[i, sl] = outv[i, sl] + slab[i, sl]

        gh = [None, None]
        oh = [None, None]
        gh[0] = start_gather(0)
        gh[1] = start_gather(1)
        for s in range(n_steps):
            g, j = s // LIN, s % LIN
            if j == 0 and oh[g % 2] is not None:
                oh[g % 2].wait()
            gh[s % 2].wait()
            accumulate(s)
            if s + 2 < n_steps:
                gh[s % 2] = start_gather(s + 2)
            if j == LIN - 1:
                oh[g % 2] = pltpu.async_copy(
                    outvs[g % 2],
                    out_h.at[pl.ds(wid * items_per_w + g * _IDX_W, _IDX_W)],
                    out_sems[g % 2])
        for h in oh:
            if h is not None:
                h.wait()

    return pool


def _mlp_body(x_ref, w1_ref, b1_ref, w2_ref, b2_ref, w3_ref, b3_ref, o_ref):
    x = x_ref[...]
    h = jnp.dot(x, w1_ref[...], preferred_element_type=jnp.float32) + b1_ref[...]
    h = jnp.maximum(h, 0.0)
    h = jnp.dot(h, w2_ref[...], preferred_element_type=jnp.float32) + b2_ref[...]
    h = jnp.maximum(h, 0.0)
    o_ref[...] = jnp.sum(h * w3_ref[...], axis=1) + b3_ref[0]


def _make_mlp(B, D, H):
    blk = 1024
    grid = (B // blk,)
    return pl.pallas_call(
        _mlp_body,
        grid=grid,
        in_specs=[
            pl.BlockSpec((blk, D), lambda i: (i, 0)),
            pl.BlockSpec((D, H), lambda i: (0, 0)),
            pl.BlockSpec((1, H), lambda i: (0, 0)),
            pl.BlockSpec((H, H), lambda i: (0, 0)),
            pl.BlockSpec((1, H), lambda i: (0, 0)),
            pl.BlockSpec((1, H), lambda i: (0, 0)),
            pl.BlockSpec(memory_space=pltpu.SMEM),
        ],
        out_specs=pl.BlockSpec((blk,), lambda i: (i,)),
        out_shape=jax.ShapeDtypeStruct((B,), jnp.float32),
    )


def kernel(player_indices, table, W1, b1, W2, b2, W3, b3):
    B, LIN = player_indices.shape
    V, D = table.shape
    H = W1.shape[1]

    idxT = player_indices.astype(jnp.int32).T + 0
    pooled = _make_pool(V, D, B, LIN)(table, idxT)

    W1s = W1 * (1.0 / LIN)
    out = _make_mlp(B, D, H)(
        pooled, W1s, b1.reshape(1, H), W2, b2.reshape(1, H),
        W3.reshape(1, H), b3)
    return out


# speedup vs baseline: 1.2874x; 1.2874x over previous
import functools

import jax
import jax.numpy as jnp
from jax import lax
from jax.experimental import pallas as pl
from jax.experimental.pallas import tpu as pltpu
from jax.experimental.pallas import tpu_sc as plsc

_NC = 2
_NS = 16
_NW = _NC * _NS
_IDX_W = 128


def _make_pool(V, D, B, LIN):
    items_per_w = B // _NW
    flat_per_w = items_per_w * LIN
    n_groups = items_per_w // _IDX_W

    mesh = plsc.VectorSubcoreMesh(core_axis_name="c", subcore_axis_name="s")

    @functools.partial(
        pl.kernel,
        out_type=jax.ShapeDtypeStruct((B, D), jnp.float32),
        mesh=mesh,
        compiler_params=pltpu.CompilerParams(use_tc_tiling_on_sc=False),
        scratch_types=[
            pltpu.VMEM((flat_per_w,), jnp.int32),
        ]
        + [pltpu.VMEM((_IDX_W, D), jnp.float32)] * 4
        + [pltpu.SemaphoreType.DMA] * 4,
    )
    def pool(table_h, idxf_h, out_h,
             idx_v, ov0, ov1, ov2, ov3, sm0, sm1, sm2, sm3):
        wid = lax.axis_index("s") * _NC + lax.axis_index("c")
        outvs = (ov0, ov1, ov2, ov3)
        sems = (sm0, sm1, sm2, sm3)

        for j in range(LIN):
            pltpu.sync_copy(
                idxf_h.at[pl.ds(j * B + wid * items_per_w, items_per_w)],
                idx_v.at[pl.ds(j * items_per_w, items_per_w)])

        def start_gather(g, j, add):
            return pltpu.async_copy(
                table_h.at[idx_v.at[pl.ds(j * items_per_w + g * _IDX_W,
                                          _IDX_W)]],
                outvs[g], sems[g], add=add)

        hs = [start_gather(g, 0, False) for g in range(n_groups)]
        for j in range(1, LIN):
            for g in range(n_groups):
                hs[g].wait()
                hs[g] = start_gather(g, j, True)
        for g in range(n_groups):
            hs[g].wait()
            hs[g] = pltpu.async_copy(
                outvs[g],
                out_h.at[pl.ds(wid * items_per_w + g * _IDX_W, _IDX_W)],
                sems[g])
        for g in range(n_groups):
            hs[g].wait()

    return pool


def _mlp_body(x_ref, w1_ref, b1_ref, w2_ref, b2_ref, w3_ref, b3_ref, o_ref):
    x = x_ref[...]
    h = jnp.dot(x, w1_ref[...], preferred_element_type=jnp.float32) + b1_ref[...]
    h = jnp.maximum(h, 0.0)
    h = jnp.dot(h, w2_ref[...], preferred_element_type=jnp.float32) + b2_ref[...]
    h = jnp.maximum(h, 0.0)
    o_ref[...] = jnp.sum(h * w3_ref[...], axis=1) + b3_ref[0]


def _make_mlp(B, D, H):
    blk = 1024
    grid = (B // blk,)
    return pl.pallas_call(
        _mlp_body,
        grid=grid,
        in_specs=[
            pl.BlockSpec((blk, D), lambda i: (i, 0)),
            pl.BlockSpec((D, H), lambda i: (0, 0)),
            pl.BlockSpec((1, H), lambda i: (0, 0)),
            pl.BlockSpec((H, H), lambda i: (0, 0)),
            pl.BlockSpec((1, H), lambda i: (0, 0)),
            pl.BlockSpec((1, H), lambda i: (0, 0)),
            pl.BlockSpec(memory_space=pltpu.SMEM),
        ],
        out_specs=pl.BlockSpec((blk,), lambda i: (i,)),
        out_shape=jax.ShapeDtypeStruct((B,), jnp.float32),
    )


def kernel(player_indices, table, W1, b1, W2, b2, W3, b3):
    B, LIN = player_indices.shape
    V, D = table.shape
    H = W1.shape[1]

    idx_flat = player_indices.astype(jnp.int32).T.reshape(LIN * B)
    pooled = _make_pool(V, D, B, LIN)(table, idx_flat)

    W1s = W1 * (1.0 / LIN)
    out = _make_mlp(B, D, H)(
        pooled, W1s, b1.reshape(1, H), W2, b2.reshape(1, H),
        W3.reshape(1, H), b3)
    return out


# speedup vs baseline: 1.3654x; 1.0605x over previous
import functools

import jax
import jax.numpy as jnp
from jax import lax
from jax.experimental import pallas as pl
from jax.experimental.pallas import tpu as pltpu
from jax.experimental.pallas import tpu_sc as plsc

_NC = 2
_NS = 16
_NW = _NC * _NS
_IDX_W = 128


def _make_pool(V, Dp, B, LIN):
    items_per_w = B // _NW
    flat_per_w = items_per_w * LIN
    n_groups = items_per_w // _IDX_W

    mesh = plsc.VectorSubcoreMesh(core_axis_name="c", subcore_axis_name="s")

    @functools.partial(
        pl.kernel,
        out_type=jax.ShapeDtypeStruct((B, Dp), jnp.float32),
        mesh=mesh,
        compiler_params=pltpu.CompilerParams(use_tc_tiling_on_sc=False),
        scratch_types=[
            pltpu.VMEM((flat_per_w,), jnp.int32),
        ]
        + [pltpu.VMEM((_IDX_W, Dp), jnp.float32)] * 4
        + [pltpu.SemaphoreType.DMA] * 4,
    )
    def pool(table_h, idxf_h, out_h,
             idx_v, ov0, ov1, ov2, ov3, sm0, sm1, sm2, sm3):
        wid = lax.axis_index("s") * _NC + lax.axis_index("c")
        outvs = (ov0, ov1, ov2, ov3)
        sems = (sm0, sm1, sm2, sm3)

        for j in range(LIN):
            pltpu.sync_copy(
                idxf_h.at[pl.ds(j * B + wid * items_per_w, items_per_w)],
                idx_v.at[pl.ds(j * items_per_w, items_per_w)])

        def start_gather(g, j, add):
            return pltpu.async_copy(
                table_h.at[idx_v.at[pl.ds(j * items_per_w + g * _IDX_W,
                                          _IDX_W)]],
                outvs[g], sems[g], add=add)

        hs = [start_gather(g, 0, False) for g in range(n_groups)]
        for j in range(1, LIN):
            for g in range(n_groups):
                hs[g].wait()
                hs[g] = start_gather(g, j, True)
        for g in range(n_groups):
            hs[g].wait()
            hs[g] = pltpu.async_copy(
                outvs[g],
                out_h.at[pl.ds(wid * items_per_w + g * _IDX_W, _IDX_W)],
                sems[g])
        for g in range(n_groups):
            hs[g].wait()

    return pool


def _mlp_body(x_ref, w1_ref, b1_ref, w2_ref, b2_ref, w3_ref, b3_ref, o_ref):
    x = x_ref[...]
    h = jnp.dot(x, w1_ref[...], preferred_element_type=jnp.float32) + b1_ref[...]
    h = jnp.maximum(h, 0.0)
    h = jnp.dot(h, w2_ref[...], preferred_element_type=jnp.float32) + b2_ref[...]
    h = jnp.maximum(h, 0.0)
    o_ref[...] = jnp.sum(h * w3_ref[...], axis=1) + b3_ref[0]


def _make_mlp(B, Dp, H):
    blk = 4096
    grid = (B // blk,)
    return pl.pallas_call(
        _mlp_body,
        grid=grid,
        in_specs=[
            pl.BlockSpec((blk, Dp), lambda i: (i, 0)),
            pl.BlockSpec((Dp, H), lambda i: (0, 0)),
            pl.BlockSpec((1, H), lambda i: (0, 0)),
            pl.BlockSpec((H, H), lambda i: (0, 0)),
            pl.BlockSpec((1, H), lambda i: (0, 0)),
            pl.BlockSpec((1, H), lambda i: (0, 0)),
            pl.BlockSpec(memory_space=pltpu.SMEM),
        ],
        out_specs=pl.BlockSpec((blk,), lambda i: (i,)),
        out_shape=jax.ShapeDtypeStruct((B,), jnp.float32),
    )


def kernel(player_indices, table, W1, b1, W2, b2, W3, b3):
    B, LIN = player_indices.shape
    V, D = table.shape
    H = W1.shape[1]
    Dp = 128

    table_p = jnp.pad(table, ((0, 0), (0, Dp - D)))

    idx_flat = player_indices.astype(jnp.int32).T.reshape(LIN * B)
    pooled = _make_pool(V, Dp, B, LIN)(table_p, idx_flat)

    W1p = jnp.pad(W1 * (1.0 / LIN), ((0, Dp - D), (0, 0)))
    out = _make_mlp(B, Dp, H)(
        pooled, W1p, b1.reshape(1, H), W2, b2.reshape(1, H),
        W3.reshape(1, H), b3)
    return out
